# Initial kernel scaffold; baseline (speedup 1.0000x reference)
import jax
import jax.numpy as jnp
from jax.experimental import pallas as pl


def kernel(embed, w_relation, triplets, labels):
    raise NotImplementedError("write your pallas kernel here")


# speedup vs baseline: 1.9105x; 1.9105x over previous
import functools

import jax
import jax.numpy as jnp
from jax import lax
from jax.experimental import pallas as pl
from jax.experimental.pallas import tpu as pltpu
from jax.experimental.pallas import tpu_sc as plsc

_T = 800000
_N = 10000
_H = 256
_R = 16
_REG = 0.01
_C = 64
_NCHUNKS = _T // _C
_NW = 32
_L = 16


def _sc_body(heads_hbm, tails_hbm, rels_hbm, embed_hbm, w_hbm, out_hbm,
             idx_h, idx_t, rels_v, s_rows, o_rows, w_rows, scores_v, sbuf,
             sem_s, sem_o, sem_w):
    wid = lax.axis_index("s") * 2 + lax.axis_index("c")
    lane = lax.broadcasted_iota(jnp.int32, (_L,), 0)

    nchunks = (_NCHUNKS - wid + _NW - 1) // _NW

    def chunk_body(i, carry):
        cid = wid + i * _NW
        base = cid * _C
        pltpu.sync_copy(heads_hbm.at[pl.ds(base, _C)], idx_h)
        pltpu.sync_copy(tails_hbm.at[pl.ds(base, _C)], idx_t)
        pltpu.sync_copy(rels_hbm.at[pl.ds(base, _C)], rels_v)
        pltpu.async_copy(embed_hbm.at[idx_h], s_rows, sem_s).wait()
        pltpu.async_copy(embed_hbm.at[idx_t], o_rows, sem_o).wait()
        pltpu.async_copy(w_hbm.at[rels_v], w_rows, sem_w).wait()

        def group_body(g, carry2):
            score_vec = jnp.zeros((_L,), jnp.float32)
            for i2 in range(_L):
                t = g * _L + i2
                acc = jnp.zeros((_L,), jnp.float32)
                for j in range(_H // _L):
                    s_j = s_rows[t, pl.ds(j * _L, _L)]
                    o_j = o_rows[t, pl.ds(j * _L, _L)]
                    w_j = w_rows[t, pl.ds(j * _L, _L)]
                    acc = acc + s_j * o_j * w_j
                for sh in (8, 4, 2, 1):
                    sbuf[pl.ds(0, _L)] = acc
                    sbuf[pl.ds(_L, _L)] = acc
                    acc = acc + sbuf[pl.ds(sh, _L)]
                score_vec = jnp.where(lane == i2, acc, score_vec)
            scores_v[pl.ds(g * _L, _L)] = score_vec
            return carry2

        lax.fori_loop(0, _C // _L, group_body, 0)
        pltpu.sync_copy(scores_v, out_hbm.at[pl.ds(base, _C)])
        return carry

    lax.fori_loop(0, nchunks, chunk_body, 0)


_sc_scores = functools.partial(
    pl.kernel,
    mesh=plsc.VectorSubcoreMesh(core_axis_name="c", subcore_axis_name="s"),
    out_type=jax.ShapeDtypeStruct((_T,), jnp.float32),
    scratch_types=[
        pltpu.VMEM((_C,), jnp.int32),
        pltpu.VMEM((_C,), jnp.int32),
        pltpu.VMEM((_C,), jnp.int32),
        pltpu.VMEM((_C, _H), jnp.float32),
        pltpu.VMEM((_C, _H), jnp.float32),
        pltpu.VMEM((_C, _H), jnp.float32),
        pltpu.VMEM((_C,), jnp.float32),
        pltpu.VMEM((2 * _L,), jnp.float32),
        pltpu.SemaphoreType.DMA,
        pltpu.SemaphoreType.DMA,
        pltpu.SemaphoreType.DMA,
    ],
)(_sc_body)


def _loss_body(s_ref, l_ref, e_ref, w_ref, out_ref):
    s = s_ref[...]
    lbl = l_ref[...]
    bce = jnp.maximum(s, 0.0) - s * lbl + jnp.log1p(jnp.exp(-jnp.abs(s)))
    predict = jnp.sum(bce) * (1.0 / _T)
    e = e_ref[...]
    w = w_ref[...]
    reg = (jnp.sum(e * e) * (1.0 / (_N * _H))
           + jnp.sum(w * w) * (1.0 / (_R * _H)))
    out_ref[...] = jnp.full((1, 1), predict + _REG * reg, jnp.float32)


def kernel(embed, w_relation, triplets, labels):
    heads = triplets[:, 0]
    rels = triplets[:, 1]
    tails = triplets[:, 2]
    scores = _sc_scores(heads, tails, rels, embed, w_relation)
    out = pl.pallas_call(
        _loss_body,
        out_shape=jax.ShapeDtypeStruct((1, 1), jnp.float32),
    )(scores.reshape(_T // 128, 128), labels.reshape(_T // 128, 128),
      embed, w_relation)
    return out[0, 0]


# speedup vs baseline: 2.2167x; 1.1603x over previous
import functools

import jax
import jax.numpy as jnp
from jax import lax
from jax.experimental import pallas as pl
from jax.experimental.pallas import tpu as pltpu
from jax.experimental.pallas import tpu_sc as plsc

_T = 800000
_N = 10000
_H = 256
_R = 16
_REG = 0.01
_C = 64
_NW = 32
_L = 16
_NCH_W = 392
_NCH = _NCH_W * _NW
_TP = _NCH * _C


def _sc_body(idx_hbm, rels_hbm, embed_hbm, w_hbm, out_hbm,
             idx_a, idx_b, rows_a, rows_b, relv_a, relv_b,
             wrows_a, wrows_b, sc_a, sc_b,
             sbuf, sem_a, sem_b, osem_a, osem_b):
    wid = lax.axis_index("s") * 2 + lax.axis_index("c")
    lane = lax.broadcasted_iota(jnp.int32, (_L,), 0)

    pltpu.sync_copy(idx_hbm.at[pl.ds(wid * 2 * _C, 2 * _C)], idx_a)
    pltpu.sync_copy(rels_hbm.at[pl.ds(wid * _C, _C)], relv_a)
    pltpu.async_copy(embed_hbm.at[idx_a], rows_a, sem_a)
    pltpu.async_copy(w_hbm.at[relv_a], wrows_a, sem_a)

    bufs = ((idx_a, rows_a, relv_a, wrows_a, sc_a, sem_a, osem_a),
            (idx_b, rows_b, relv_b, wrows_b, sc_b, sem_b, osem_b))

    def half_step(k, cur, nxt):
        idx_c, rows_c, relv_c, wrows_c, sc_c, sem_c, osem_c = cur
        idx_n, rows_n, relv_n, wrows_n, _, sem_n, _ = nxt
        cid = wid + k * _NW

        @pl.when(k + 1 < _NCH_W)
        def _():
            ncid = wid + (k + 1) * _NW
            pltpu.sync_copy(idx_hbm.at[pl.ds(ncid * 2 * _C, 2 * _C)], idx_n)
            pltpu.sync_copy(rels_hbm.at[pl.ds(ncid * _C, _C)], relv_n)
            pltpu.async_copy(embed_hbm.at[idx_n], rows_n, sem_n)
            pltpu.async_copy(w_hbm.at[relv_n], wrows_n, sem_n)

        pltpu.make_async_copy(embed_hbm.at[pl.ds(0, 2 * _C)], rows_c,
                              sem_c).wait()
        pltpu.make_async_copy(embed_hbm.at[pl.ds(0, _C)], wrows_c,
                              sem_c).wait()

        def group_body(g, carry):
            score_vec = jnp.zeros((_L,), jnp.float32)
            for i2 in range(_L):
                t = g * _L + i2
                acc = jnp.zeros((_L,), jnp.float32)
                for j in range(_H // _L):
                    s_j = rows_c[t, pl.ds(j * _L, _L)]
                    o_j = rows_c[_C + t, pl.ds(j * _L, _L)]
                    w_j = wrows_c[t, pl.ds(j * _L, _L)]
                    acc = acc + s_j * o_j * w_j
                for sh in (8, 4, 2, 1):
                    sbuf[pl.ds(0, _L)] = acc
                    sbuf[pl.ds(_L, _L)] = acc
                    acc = acc + sbuf[pl.ds(sh, _L)]
                score_vec = jnp.where(lane == i2, acc, score_vec)
            sc_c[pl.ds(g * _L, _L)] = score_vec
            return carry

        lax.fori_loop(0, _C // _L, group_body, 0)

        @pl.when(k >= 2)
        def _():
            pltpu.make_async_copy(sc_c, out_hbm.at[pl.ds(0, _C)],
                                  osem_c).wait()
        pltpu.async_copy(sc_c, out_hbm.at[pl.ds(cid * _C, _C)], osem_c)

    def pair_body(p, carry):
        half_step(2 * p, bufs[0], bufs[1])
        half_step(2 * p + 1, bufs[1], bufs[0])
        return carry

    lax.fori_loop(0, _NCH_W // 2, pair_body, 0)

    pltpu.make_async_copy(sc_a, out_hbm.at[pl.ds(0, _C)], osem_a).wait()
    pltpu.make_async_copy(sc_b, out_hbm.at[pl.ds(0, _C)], osem_b).wait()


_sc_scores = functools.partial(
    pl.kernel,
    mesh=plsc.VectorSubcoreMesh(core_axis_name="c", subcore_axis_name="s"),
    out_type=jax.ShapeDtypeStruct((_TP,), jnp.float32),
    scratch_types=[
        pltpu.VMEM((2 * _C,), jnp.int32),
        pltpu.VMEM((2 * _C,), jnp.int32),
        pltpu.VMEM((2 * _C, _H), jnp.float32),
        pltpu.VMEM((2 * _C, _H), jnp.float32),
        pltpu.VMEM((_C,), jnp.int32),
        pltpu.VMEM((_C,), jnp.int32),
        pltpu.VMEM((_C, _H), jnp.float32),
        pltpu.VMEM((_C, _H), jnp.float32),
        pltpu.VMEM((_C,), jnp.float32),
        pltpu.VMEM((_C,), jnp.float32),
        pltpu.VMEM((2 * _L,), jnp.float32),
        pltpu.SemaphoreType.DMA,
        pltpu.SemaphoreType.DMA,
        pltpu.SemaphoreType.DMA,
        pltpu.SemaphoreType.DMA,
    ],
)(_sc_body)


def _loss_body(s_ref, l_ref, e_ref, w_ref, out_ref):
    s = s_ref[...]
    lbl = l_ref[...]
    bce = jnp.maximum(s, 0.0) - s * lbl + jnp.log1p(jnp.exp(-jnp.abs(s)))
    predict = jnp.sum(bce) * (1.0 / _T)
    e = e_ref[...]
    w = w_ref[...]
    reg = (jnp.sum(e * e) * (1.0 / (_N * _H))
           + jnp.sum(w * w) * (1.0 / (_R * _H)))
    out_ref[...] = jnp.full((1, 1), predict + _REG * reg, jnp.float32)


def kernel(embed, w_relation, triplets, labels):
    heads = triplets[:, 0]
    rels = triplets[:, 1]
    tails = triplets[:, 2]
    z = jnp.zeros((_TP - _T,), jnp.int32)
    heads_p = jnp.concatenate([heads, z]).reshape(_NCH, _C)
    tails_p = jnp.concatenate([tails, z]).reshape(_NCH, _C)
    rels_p = jnp.concatenate([rels, z])
    idx_ht = jnp.concatenate([heads_p, tails_p], axis=1).reshape(-1)
    scores = _sc_scores(idx_ht, rels_p, embed, w_relation)[:_T]
    out = pl.pallas_call(
        _loss_body,
        out_shape=jax.ShapeDtypeStruct((1, 1), jnp.float32),
    )(scores.reshape(_T // 128, 128), labels.reshape(_T // 128, 128),
      embed, w_relation)
    return out[0, 0]


# speedup vs baseline: 3.2732x; 1.4766x over previous
import functools

import jax
import jax.numpy as jnp
from jax import lax
from jax.experimental import pallas as pl
from jax.experimental.pallas import tpu as pltpu
from jax.experimental.pallas import tpu_sc as plsc

_T = 800000
_N = 10000
_H = 256
_R = 16
_REG = 0.01
_C = 64
_NW = 32
_L = 16
_NCH_W = 392
_NCH = _NCH_W * _NW
_TP = _NCH * _C
_WREP = 256


def _sc_body(idx_hbm, rels_hbm, embed_hbm, w_hbm, out_hbm,
             idx_a, idx_b, rows_a, rows_b, relv_a, relv_b,
             wrows_a, wrows_b, sc_a, sc_b,
             sbuf, sem_a, sem_b, osem_a, osem_b):
    wid = lax.axis_index("s") * 2 + lax.axis_index("c")
    lane = lax.broadcasted_iota(jnp.int32, (_L,), 0)

    pltpu.sync_copy(idx_hbm.at[pl.ds(wid * 2 * _C, 2 * _C)], idx_a)
    pltpu.sync_copy(rels_hbm.at[pl.ds(wid * _C, _C)], relv_a)
    pltpu.async_copy(embed_hbm.at[idx_a], rows_a, sem_a)
    pltpu.async_copy(w_hbm.at[relv_a], wrows_a, sem_a)

    bufs = ((idx_a, rows_a, relv_a, wrows_a, sc_a, sem_a, osem_a),
            (idx_b, rows_b, relv_b, wrows_b, sc_b, sem_b, osem_b))

    def half_step(k, cur, nxt):
        idx_c, rows_c, relv_c, wrows_c, sc_c, sem_c, osem_c = cur
        idx_n, rows_n, relv_n, wrows_n, _, sem_n, _ = nxt
        cid = wid + k * _NW

        @pl.when(k + 1 < _NCH_W)
        def _():
            ncid = wid + (k + 1) * _NW
            pltpu.sync_copy(idx_hbm.at[pl.ds(ncid * 2 * _C, 2 * _C)], idx_n)
            pltpu.sync_copy(rels_hbm.at[pl.ds(ncid * _C, _C)], relv_n)
            pltpu.async_copy(embed_hbm.at[idx_n], rows_n, sem_n)
            pltpu.async_copy(w_hbm.at[relv_n], wrows_n, sem_n)

        pltpu.make_async_copy(embed_hbm.at[pl.ds(0, 2 * _C)], rows_c,
                              sem_c).wait()
        pltpu.make_async_copy(embed_hbm.at[pl.ds(0, _C)], wrows_c,
                              sem_c).wait()

        def group_body(g, carry):
            score_vec = jnp.zeros((_L,), jnp.float32)
            for i2 in range(_L):
                t = g * _L + i2
                acc = jnp.zeros((_L,), jnp.float32)
                for j in range(_H // _L):
                    s_j = rows_c[t, pl.ds(j * _L, _L)]
                    o_j = rows_c[_C + t, pl.ds(j * _L, _L)]
                    w_j = wrows_c[t, pl.ds(j * _L, _L)]
                    acc = acc + s_j * o_j * w_j
                for sh in (8, 4, 2, 1):
                    sbuf[pl.ds(0, _L)] = acc
                    sbuf[pl.ds(_L, _L)] = acc
                    acc = acc + sbuf[pl.ds(sh, _L)]
                score_vec = jnp.where(lane == i2, acc, score_vec)
            sc_c[pl.ds(g * _L, _L)] = score_vec
            return carry

        lax.fori_loop(0, _C // _L, group_body, 0)

        @pl.when(k >= 2)
        def _():
            pltpu.make_async_copy(sc_c, out_hbm.at[pl.ds(0, _C)],
                                  osem_c).wait()
        pltpu.async_copy(sc_c, out_hbm.at[pl.ds(cid * _C, _C)], osem_c)

    def pair_body(p, carry):
        half_step(2 * p, bufs[0], bufs[1])
        half_step(2 * p + 1, bufs[1], bufs[0])
        return carry

    lax.fori_loop(0, _NCH_W // 2, pair_body, 0)

    pltpu.make_async_copy(sc_a, out_hbm.at[pl.ds(0, _C)], osem_a).wait()
    pltpu.make_async_copy(sc_b, out_hbm.at[pl.ds(0, _C)], osem_b).wait()


_sc_scores = functools.partial(
    pl.kernel,
    mesh=plsc.VectorSubcoreMesh(core_axis_name="c", subcore_axis_name="s"),
    out_type=jax.ShapeDtypeStruct((_TP,), jnp.float32),
    scratch_types=[
        pltpu.VMEM((2 * _C,), jnp.int32),
        pltpu.VMEM((2 * _C,), jnp.int32),
        pltpu.VMEM((2 * _C, _H), jnp.float32),
        pltpu.VMEM((2 * _C, _H), jnp.float32),
        pltpu.VMEM((_C,), jnp.int32),
        pltpu.VMEM((_C,), jnp.int32),
        pltpu.VMEM((_C, _H), jnp.float32),
        pltpu.VMEM((_C, _H), jnp.float32),
        pltpu.VMEM((_C,), jnp.float32),
        pltpu.VMEM((_C,), jnp.float32),
        pltpu.VMEM((2 * _L,), jnp.float32),
        pltpu.SemaphoreType.DMA,
        pltpu.SemaphoreType.DMA,
        pltpu.SemaphoreType.DMA,
        pltpu.SemaphoreType.DMA,
    ],
)(_sc_body)


def _loss_body(s_ref, l_ref, e_ref, w_ref, out_ref):
    s = s_ref[...]
    lbl = l_ref[...]
    bce = jnp.maximum(s, 0.0) - s * lbl + jnp.log1p(jnp.exp(-jnp.abs(s)))
    predict = jnp.sum(bce) * (1.0 / _T)
    e = e_ref[...]
    w = w_ref[...]
    reg = (jnp.sum(e * e) * (1.0 / (_N * _H))
           + jnp.sum(w * w) * (1.0 / (_R * _H)))
    out_ref[...] = jnp.full((1, 1), predict + _REG * reg, jnp.float32)


def kernel(embed, w_relation, triplets, labels):
    heads = triplets[:, 0]
    rels = triplets[:, 1]
    tails = triplets[:, 2]
    z = jnp.zeros((_TP - _T,), jnp.int32)
    heads_p = jnp.concatenate([heads, z]).reshape(_NCH, _C)
    tails_p = jnp.concatenate([tails, z]).reshape(_NCH, _C)
    rels_p = jnp.concatenate([rels, z])
    mix = (jnp.arange(_TP, dtype=jnp.uint32) * jnp.uint32(2654435761)
           ).astype(jnp.int32) & (_WREP - 1)
    widx = mix * _R + rels_p
    w_rep = jnp.tile(w_relation, (_WREP, 1))
    idx_ht = jnp.concatenate([heads_p, tails_p], axis=1).reshape(-1)
    scores = _sc_scores(idx_ht, widx, embed, w_rep)[:_T]
    out = pl.pallas_call(
        _loss_body,
        out_shape=jax.ShapeDtypeStruct((1, 1), jnp.float32),
    )(scores.reshape(_T // 128, 128), labels.reshape(_T // 128, 128),
      embed, w_relation)
    return out[0, 0]
